# baseline (speedup 1.0000x reference)
import functools

import jax
import jax.numpy as jnp
from jax import lax
from jax.experimental import pallas as pl
from jax.experimental.pallas import tpu as pltpu
from jax.experimental.pallas import tpu_sc as plsc

N = 10000
E = 320000
F = 128
HID = 64
SOUT = 13
SP = 16

NC = 2
NS = 16
NW = NC * NS
LANES = 16

CHUNK = 128
G = 5
CPT = 80
NG = CPT // G
EP = NW * CPT * CHUNK
NROWS = EP // CHUNK
NPAD = 10240
NPT = NPAD // NS

@functools.lru_cache(maxsize=1)
def _mesh():
    return plsc.VectorSubcoreMesh(core_axis_name="c", subcore_axis_name="s",
                                  num_cores=NC, num_subcores=NS)


def _deg_body(dst_hbm, ew_hbm, out_hbm, acc, dstb, ewb, zbuf, ssem0):
    c = lax.axis_index("c")
    s = lax.axis_index("s")
    w = s * NC + c
    ssems = (ssem0,)

    zero = jnp.zeros((LANES,), jnp.float32)
    for i in range(NPT // LANES):
        zbuf[pl.ds(i * LANES, LANES)] = zero
    pltpu.sync_copy(zbuf, acc.at[pl.ds(s * NPT, NPT)])
    plsc.subcore_barrier()

    def load_idx(grp, sl):
        row0 = w * CPT + grp * G
        pltpu.sync_copy(dst_hbm.at[pl.ds(row0, G)], dstb.at[sl])
        pltpu.sync_copy(ew_hbm.at[pl.ds(row0, G)], ewb.at[sl])

    def fire(sl):
        for k in range(G):
            pltpu.make_async_copy(
                ewb.at[sl, k], acc.at[dstb.at[sl, k]],
                ssems[sl]).start(add=True)

    def drain(sl):
        for k in range(G):
            pltpu.make_async_copy(
                ewb.at[sl, k], acc.at[dstb.at[sl, k]], ssems[sl]).wait()

    def step(j, carry):
        load_idx(j, 0)
        fire(0)
        drain(0)
        return carry

    lax.fori_loop(0, NG, step, 0)
    plsc.subcore_barrier()
    pltpu.sync_copy(acc.at[pl.ds(s * NPT, NPT)],
                    out_hbm.at[c, pl.ds(s * NPT, NPT)])


def _sc_degree(dstp, ewp):
    return pl.kernel(
        _deg_body,
        out_type=jax.ShapeDtypeStruct((NC, NPAD), jnp.float32),
        mesh=_mesh(),
        compiler_params=pltpu.CompilerParams(use_tc_tiling_on_sc=False),
        scratch_types=[
            pltpu.VMEM_SHARED((NPAD,), jnp.float32),
            pltpu.VMEM((2, G, CHUNK), jnp.int32),
            pltpu.VMEM((2, G, CHUNK), jnp.float32),
            pltpu.VMEM((NPT,), jnp.float32),
            pltpu.SemaphoreType.DMA,
        ],
    )(dstp, ewp)


def _agg_body(width, gsz, src_hbm, dst_hbm, ew_hbm, g_hbm, out_hbm,
              acc, rows, srcb, dstb, ewb,
              gsem0, gsem1, ssem0, ssem1):
    G = gsz
    NG = CPT // G
    c = lax.axis_index("c")
    s = lax.axis_index("s")
    w = s * NC + c
    q = width // LANES
    gsems = (gsem0, gsem1)
    ssems = (ssem0, ssem1)


    zero = jnp.zeros((LANES,), jnp.float32)
    for i in range(CHUNK):
        for k in range(q):
            rows[0, 0, i, pl.ds(k * LANES, LANES)] = zero
    for i in range(NPT // CHUNK):
        pltpu.sync_copy(rows.at[0, 0],
                        acc.at[pl.ds(s * NPT + i * CHUNK, CHUNK)])
    plsc.subcore_barrier()

    def load_idx(grp, sl):
        row0 = w * CPT + grp * G
        pltpu.sync_copy(src_hbm.at[pl.ds(row0, G)], srcb.at[sl])
        pltpu.sync_copy(dst_hbm.at[pl.ds(row0, G)], dstb.at[sl])
        pltpu.sync_copy(ew_hbm.at[pl.ds(row0, G)], ewb.at[sl])

    def fire_gathers(sl):
        for k in range(G):
            pltpu.make_async_copy(
                g_hbm.at[srcb.at[sl, k]], rows.at[sl, k], gsems[sl]).start()

    def drain_gathers(sl):
        for k in range(G):
            pltpu.make_async_copy(
                g_hbm.at[srcb.at[sl, k]], rows.at[sl, k], gsems[sl]).wait()

    def fire_scatters(sl):
        for k in range(G):
            pltpu.make_async_copy(
                rows.at[sl, k], acc.at[dstb.at[sl, k]],
                ssems[sl]).start(add=True)

    def drain_scatters(sl):
        for k in range(G):
            pltpu.make_async_copy(
                rows.at[sl, k], acc.at[dstb.at[sl, k]], ssems[sl]).wait()

    def scale(sl):
        for k in range(G):
            def mk(_k):
                def body(e16):
                    base = e16 * LANES
                    ws = ewb[sl, _k, pl.ds(base, LANES)]
                    for i in range(LANES):
                        sc = ws[i]
                        for kk in range(q):
                            cs = pl.ds(kk * LANES, LANES)
                            rows[sl, _k, base + i, cs] = (
                                rows[sl, _k, base + i, cs] * sc)
                return body

            plsc.parallel_loop(0, CHUNK // LANES, unroll=2)(mk(k))

    load_idx(0, 0)
    fire_gathers(0)
    load_idx(1, 1)
    fire_gathers(1)
    drain_gathers(0)
    scale(0)
    fire_scatters(0)

    def pair(p, carry):
        drain_scatters(0)
        load_idx(2 * p, 0)
        fire_gathers(0)
        drain_gathers(1)
        scale(1)
        fire_scatters(1)
        drain_scatters(1)
        load_idx(2 * p + 1, 1)
        fire_gathers(1)
        drain_gathers(0)
        scale(0)
        fire_scatters(0)
        return carry

    lax.fori_loop(1, NG // 2, pair, 0)
    drain_gathers(1)
    scale(1)
    fire_scatters(1)
    drain_scatters(0)
    drain_scatters(1)
    plsc.subcore_barrier()
    pltpu.sync_copy(acc.at[pl.ds(s * NPT, NPT)],
                    out_hbm.at[c, pl.ds(s * NPT, NPT)])


def _sc_aggregate(srcp, dstp, ewp, g):
    width = g.shape[1]
    gsz = 5 if width > 16 else 10
    return pl.kernel(
        functools.partial(_agg_body, width, gsz),
        out_type=jax.ShapeDtypeStruct((NC, NPAD, width), jnp.float32),
        mesh=_mesh(),
        compiler_params=pltpu.CompilerParams(use_tc_tiling_on_sc=False),
        scratch_types=[
            pltpu.VMEM_SHARED((NPAD, width), jnp.float32),
            pltpu.VMEM((2, gsz, CHUNK, width), jnp.float32),
            pltpu.VMEM((2, gsz, CHUNK), jnp.int32),
            pltpu.VMEM((2, gsz, CHUNK), jnp.int32),
            pltpu.VMEM((2, gsz, CHUNK), jnp.float32),
            pltpu.SemaphoreType.DMA,
            pltpu.SemaphoreType.DMA,
            pltpu.SemaphoreType.DMA,
            pltpu.SemaphoreType.DMA,
        ],
    )(srcp, dstp, ewp, g)


def _dinv_from_parts(deg_ref):
    deg = deg_ref[0, :N] + deg_ref[1, :N] + 1.0
    return jnp.where(deg > 0.0, lax.rsqrt(deg), 0.0)


def _tc_pre_body(x_ref, w_ref, deg_ref, h_ref, g_ref):
    h = jnp.dot(x_ref[...], w_ref[...], preferred_element_type=jnp.float32)
    dinv = _dinv_from_parts(deg_ref)
    h_ref[...] = h
    g_ref[...] = h * dinv[:, None]


def _tc_layer_body(relu, deg_ref, acc_ref, h_ref, b_ref, w_ref,
                   h2_ref, g2_ref):
    dinv = _dinv_from_parts(deg_ref)
    acc = acc_ref[0, :N, :] + acc_ref[1, :N, :]
    z = acc * dinv[:, None] + h_ref[...] * (dinv * dinv)[:, None] + b_ref[...]
    if relu:
        z = jnp.maximum(z, 0.0)
    h2 = jnp.dot(z, w_ref[...], preferred_element_type=jnp.float32)
    h2_ref[...] = h2
    g2_ref[...] = h2 * dinv[:, None]


def _tc_final_body(deg_ref, acc_ref, h_ref, b_ref, out_ref):
    dinv = _dinv_from_parts(deg_ref)
    acc = acc_ref[0, :N, :] + acc_ref[1, :N, :]
    out_ref[...] = (acc * dinv[:, None]
                    + h_ref[...] * (dinv * dinv)[:, None] + b_ref[...])


def _tc_pre(x, w1, degp):
    return pl.pallas_call(
        _tc_pre_body,
        out_shape=(jax.ShapeDtypeStruct((N, HID), jnp.float32),
                   jax.ShapeDtypeStruct((N, HID), jnp.float32)),
    )(x, w1, degp)


def _tc_layer(degp, accp, h, b, w, relu=True):
    wout = w.shape[1]
    return pl.pallas_call(
        functools.partial(_tc_layer_body, relu),
        out_shape=(jax.ShapeDtypeStruct((N, wout), jnp.float32),
                   jax.ShapeDtypeStruct((N, wout), jnp.float32)),
    )(degp, accp, h, b, w)


def _tc_final(degp, accp, h, b):
    return pl.pallas_call(
        _tc_final_body,
        out_shape=jax.ShapeDtypeStruct((N, SP), jnp.float32),
    )(degp, accp, h, b)


@jax.jit
def kernel(x, edge_index, edge_weight, W1, b1, W2, b2, W3, b3):
    src = edge_index[0]
    dst = edge_index[1]

    pad = EP - E
    spread = (jnp.arange(pad, dtype=jnp.int32) * 37) % N
    srcp = jnp.concatenate([src, spread]).reshape(NROWS, CHUNK)
    dstp = jnp.concatenate([dst, spread]).reshape(NROWS, CHUNK)
    ewp = jnp.concatenate(
        [edge_weight, jnp.zeros((pad,), jnp.float32)]).reshape(NROWS, CHUNK)

    w3p = jnp.zeros((HID, SP), jnp.float32).at[:, :SOUT].set(W3)
    b3p = jnp.zeros((SP,), jnp.float32).at[:SOUT].set(b3)

    degp = _sc_degree(dstp, ewp)
    h1, g1 = _tc_pre(x, W1, degp)

    acc1 = _sc_aggregate(srcp, dstp, ewp, g1)
    h2, g2 = _tc_layer(degp, acc1, h1, b1.reshape(1, HID), W2, relu=True)

    acc2 = _sc_aggregate(srcp, dstp, ewp, g2)
    h3, g3 = _tc_layer(degp, acc2, h2, b2.reshape(1, HID), w3p, relu=True)

    acc3 = _sc_aggregate(srcp, dstp, ewp, g3)
    out = _tc_final(degp, acc3, h3, b3p.reshape(1, SP))

    return out[:, :SOUT]


# speedup vs baseline: 1.0118x; 1.0118x over previous
import functools

import jax
import jax.numpy as jnp
from jax import lax
from jax.experimental import pallas as pl
from jax.experimental.pallas import tpu as pltpu
from jax.experimental.pallas import tpu_sc as plsc

N = 10000
E = 320000
F = 128
HID = 64
SOUT = 13
SP = 16

NC = 2
NS = 16
NW = NC * NS
LANES = 16

CHUNK = 128
G = 5
CPT = 80
NG = CPT // G
EP = NW * CPT * CHUNK
NROWS = EP // CHUNK
NPAD = 10240
NPT = NPAD // NS

@functools.lru_cache(maxsize=1)
def _mesh():
    return plsc.VectorSubcoreMesh(core_axis_name="c", subcore_axis_name="s",
                                  num_cores=NC, num_subcores=NS)


def _deg_body(dst_hbm, ew_hbm, out_hbm, acc, dstb, ewb, zbuf, ssem0, ssem1):
    c = lax.axis_index("c")
    s = lax.axis_index("s")
    w = s * NC + c
    ssems = (ssem0, ssem1)

    zero = jnp.zeros((LANES,), jnp.float32)
    for i in range(NPT // LANES):
        zbuf[pl.ds(i * LANES, LANES)] = zero
    pltpu.sync_copy(zbuf, acc.at[pl.ds(s * NPT, NPT)])
    plsc.subcore_barrier()

    def load_idx(grp, sl):
        row0 = w * CPT + grp * G
        pltpu.sync_copy(dst_hbm.at[pl.ds(row0, G)], dstb.at[sl])
        pltpu.sync_copy(ew_hbm.at[pl.ds(row0, G)], ewb.at[sl])

    def fire(sl):
        for k in range(G):
            pltpu.make_async_copy(
                ewb.at[sl, k], acc.at[dstb.at[sl, k]],
                ssems[sl]).start(add=True)

    def drain(sl):
        for k in range(G):
            pltpu.make_async_copy(
                ewb.at[sl, k], acc.at[dstb.at[sl, k]], ssems[sl]).wait()

    load_idx(0, 0)
    fire(0)

    def pair(p, carry):
        load_idx(2 * p - 1, 1)
        fire(1)
        drain(0)
        load_idx(2 * p, 0)
        fire(0)
        drain(1)
        return carry

    lax.fori_loop(1, NG // 2, pair, 0)
    load_idx(NG - 1, 1)
    fire(1)
    drain(0)
    drain(1)
    plsc.subcore_barrier()
    pltpu.sync_copy(acc.at[pl.ds(s * NPT, NPT)],
                    out_hbm.at[c, pl.ds(s * NPT, NPT)])


def _sc_degree(dstp, ewp):
    return pl.kernel(
        _deg_body,
        out_type=jax.ShapeDtypeStruct((NC, NPAD), jnp.float32),
        mesh=_mesh(),
        compiler_params=pltpu.CompilerParams(use_tc_tiling_on_sc=False),
        scratch_types=[
            pltpu.VMEM_SHARED((NPAD,), jnp.float32),
            pltpu.VMEM((2, G, CHUNK), jnp.int32),
            pltpu.VMEM((2, G, CHUNK), jnp.float32),
            pltpu.VMEM((NPT,), jnp.float32),
            pltpu.SemaphoreType.DMA,
            pltpu.SemaphoreType.DMA,
        ],
    )(dstp, ewp)


def _agg_body(width, gsz, src_hbm, dst_hbm, ew_hbm, g_hbm, out_hbm,
              acc, rows, srcb, dstb, ewb,
              gsem0, gsem1, ssem0, ssem1):
    G = gsz
    NG = CPT // G
    c = lax.axis_index("c")
    s = lax.axis_index("s")
    w = s * NC + c
    q = width // LANES
    gsems = (gsem0, gsem1)
    ssems = (ssem0, ssem1)


    zero = jnp.zeros((LANES,), jnp.float32)
    for i in range(CHUNK):
        for k in range(q):
            rows[0, 0, i, pl.ds(k * LANES, LANES)] = zero
    for i in range(NPT // CHUNK):
        pltpu.sync_copy(rows.at[0, 0],
                        acc.at[pl.ds(s * NPT + i * CHUNK, CHUNK)])
    plsc.subcore_barrier()

    def load_idx(grp, sl):
        row0 = w * CPT + grp * G
        pltpu.sync_copy(src_hbm.at[pl.ds(row0, G)], srcb.at[sl])
        pltpu.sync_copy(dst_hbm.at[pl.ds(row0, G)], dstb.at[sl])
        pltpu.sync_copy(ew_hbm.at[pl.ds(row0, G)], ewb.at[sl])

    def fire_gathers(sl):
        for k in range(G):
            pltpu.make_async_copy(
                g_hbm.at[srcb.at[sl, k]], rows.at[sl, k], gsems[sl]).start()

    def drain_gathers(sl):
        for k in range(G):
            pltpu.make_async_copy(
                g_hbm.at[srcb.at[sl, k]], rows.at[sl, k], gsems[sl]).wait()

    def fire_scatters(sl):
        for k in range(G):
            pltpu.make_async_copy(
                rows.at[sl, k], acc.at[dstb.at[sl, k]],
                ssems[sl]).start(add=True)

    def drain_scatters(sl):
        for k in range(G):
            pltpu.make_async_copy(
                rows.at[sl, k], acc.at[dstb.at[sl, k]], ssems[sl]).wait()

    def scale(sl):
        for k in range(G):
            def mk(_k):
                def body(e16):
                    base = e16 * LANES
                    ws = ewb[sl, _k, pl.ds(base, LANES)]
                    for i in range(LANES):
                        sc = ws[i]
                        for kk in range(q):
                            cs = pl.ds(kk * LANES, LANES)
                            rows[sl, _k, base + i, cs] = (
                                rows[sl, _k, base + i, cs] * sc)
                return body

            plsc.parallel_loop(0, CHUNK // LANES)(mk(k))

    load_idx(0, 0)
    fire_gathers(0)
    load_idx(1, 1)
    fire_gathers(1)
    drain_gathers(0)
    scale(0)
    fire_scatters(0)

    def pair(p, carry):
        drain_scatters(0)
        load_idx(2 * p, 0)
        fire_gathers(0)
        drain_gathers(1)
        scale(1)
        fire_scatters(1)
        drain_scatters(1)
        load_idx(2 * p + 1, 1)
        fire_gathers(1)
        drain_gathers(0)
        scale(0)
        fire_scatters(0)
        return carry

    lax.fori_loop(1, NG // 2, pair, 0)
    drain_gathers(1)
    scale(1)
    fire_scatters(1)
    drain_scatters(0)
    drain_scatters(1)
    plsc.subcore_barrier()
    pltpu.sync_copy(acc.at[pl.ds(s * NPT, NPT)],
                    out_hbm.at[c, pl.ds(s * NPT, NPT)])


def _sc_aggregate(srcp, dstp, ewp, g):
    width = g.shape[1]
    gsz = 5 if width > 16 else 10
    return pl.kernel(
        functools.partial(_agg_body, width, gsz),
        out_type=jax.ShapeDtypeStruct((NC, NPAD, width), jnp.float32),
        mesh=_mesh(),
        compiler_params=pltpu.CompilerParams(use_tc_tiling_on_sc=False),
        scratch_types=[
            pltpu.VMEM_SHARED((NPAD, width), jnp.float32),
            pltpu.VMEM((2, gsz, CHUNK, width), jnp.float32),
            pltpu.VMEM((2, gsz, CHUNK), jnp.int32),
            pltpu.VMEM((2, gsz, CHUNK), jnp.int32),
            pltpu.VMEM((2, gsz, CHUNK), jnp.float32),
            pltpu.SemaphoreType.DMA,
            pltpu.SemaphoreType.DMA,
            pltpu.SemaphoreType.DMA,
            pltpu.SemaphoreType.DMA,
        ],
    )(srcp, dstp, ewp, g)


def _dinv_from_parts(deg_ref):
    deg = deg_ref[0, :N] + deg_ref[1, :N] + 1.0
    return jnp.where(deg > 0.0, lax.rsqrt(deg), 0.0)


def _tc_pre_body(x_ref, w_ref, deg_ref, h_ref, g_ref):
    h = jnp.dot(x_ref[...], w_ref[...], preferred_element_type=jnp.float32)
    dinv = _dinv_from_parts(deg_ref)
    h_ref[...] = h
    g_ref[...] = h * dinv[:, None]


def _tc_layer_body(relu, deg_ref, acc_ref, h_ref, b_ref, w_ref,
                   h2_ref, g2_ref):
    dinv = _dinv_from_parts(deg_ref)
    acc = acc_ref[0, :N, :] + acc_ref[1, :N, :]
    z = acc * dinv[:, None] + h_ref[...] * (dinv * dinv)[:, None] + b_ref[...]
    if relu:
        z = jnp.maximum(z, 0.0)
    h2 = jnp.dot(z, w_ref[...], preferred_element_type=jnp.float32)
    h2_ref[...] = h2
    g2_ref[...] = h2 * dinv[:, None]


def _tc_final_body(deg_ref, acc_ref, h_ref, b_ref, out_ref):
    dinv = _dinv_from_parts(deg_ref)
    acc = acc_ref[0, :N, :] + acc_ref[1, :N, :]
    out_ref[...] = (acc * dinv[:, None]
                    + h_ref[...] * (dinv * dinv)[:, None] + b_ref[...])


def _tc_pre(x, w1, degp):
    return pl.pallas_call(
        _tc_pre_body,
        out_shape=(jax.ShapeDtypeStruct((N, HID), jnp.float32),
                   jax.ShapeDtypeStruct((N, HID), jnp.float32)),
    )(x, w1, degp)


def _tc_layer(degp, accp, h, b, w, relu=True):
    wout = w.shape[1]
    return pl.pallas_call(
        functools.partial(_tc_layer_body, relu),
        out_shape=(jax.ShapeDtypeStruct((N, wout), jnp.float32),
                   jax.ShapeDtypeStruct((N, wout), jnp.float32)),
    )(degp, accp, h, b, w)


def _tc_final(degp, accp, h, b):
    return pl.pallas_call(
        _tc_final_body,
        out_shape=jax.ShapeDtypeStruct((N, SP), jnp.float32),
    )(degp, accp, h, b)


@jax.jit
def kernel(x, edge_index, edge_weight, W1, b1, W2, b2, W3, b3):
    src = edge_index[0]
    dst = edge_index[1]

    pad = EP - E
    spread = (jnp.arange(pad, dtype=jnp.int32) * 37) % N
    srcp = jnp.concatenate([src, spread]).reshape(NROWS, CHUNK)
    dstp = jnp.concatenate([dst, spread]).reshape(NROWS, CHUNK)
    ewp = jnp.concatenate(
        [edge_weight, jnp.zeros((pad,), jnp.float32)]).reshape(NROWS, CHUNK)

    w3p = jnp.zeros((HID, SP), jnp.float32).at[:, :SOUT].set(W3)
    b3p = jnp.zeros((SP,), jnp.float32).at[:SOUT].set(b3)

    degp = _sc_degree(dstp, ewp)
    h1, g1 = _tc_pre(x, W1, degp)

    acc1 = _sc_aggregate(srcp, dstp, ewp, g1)
    h2, g2 = _tc_layer(degp, acc1, h1, b1.reshape(1, HID), W2, relu=True)

    acc2 = _sc_aggregate(srcp, dstp, ewp, g2)
    h3, g3 = _tc_layer(degp, acc2, h2, b2.reshape(1, HID), w3p, relu=True)

    acc3 = _sc_aggregate(srcp, dstp, ewp, g3)
    out = _tc_final(degp, acc3, h3, b3p.reshape(1, SP))

    return out[:, :SOUT]


# speedup vs baseline: 1.0301x; 1.0181x over previous
import functools

import jax
import jax.numpy as jnp
from jax import lax
from jax.experimental import pallas as pl
from jax.experimental.pallas import tpu as pltpu
from jax.experimental.pallas import tpu_sc as plsc

N = 10000
E = 320000
F = 128
HID = 64
SOUT = 13
SP = 16

NC = 2
NS = 16
NW = NC * NS
LANES = 16

CHUNK = 128
G = 10
CPT = 80
NG = CPT // G
EP = NW * CPT * CHUNK
NROWS = EP // CHUNK
NPAD = 10240
NPT = NPAD // NS

@functools.lru_cache(maxsize=1)
def _mesh():
    return plsc.VectorSubcoreMesh(core_axis_name="c", subcore_axis_name="s",
                                  num_cores=NC, num_subcores=NS)


def _deg_body(dst_hbm, ew_hbm, out_hbm, acc, dstb, ewb, zbuf, ssem0, ssem1):
    c = lax.axis_index("c")
    s = lax.axis_index("s")
    w = s * NC + c
    ssems = (ssem0, ssem1)

    zero = jnp.zeros((LANES,), jnp.float32)
    for i in range(NPT // LANES):
        zbuf[pl.ds(i * LANES, LANES)] = zero
    pltpu.sync_copy(zbuf, acc.at[pl.ds(s * NPT, NPT)])
    plsc.subcore_barrier()

    def load_idx(grp, sl):
        row0 = w * CPT + grp * G
        pltpu.sync_copy(dst_hbm.at[pl.ds(row0, G)], dstb.at[sl])
        pltpu.sync_copy(ew_hbm.at[pl.ds(row0, G)], ewb.at[sl])

    def fire(sl):
        for k in range(G):
            pltpu.make_async_copy(
                ewb.at[sl, k], acc.at[dstb.at[sl, k]],
                ssems[sl]).start(add=True)

    def drain(sl):
        for k in range(G):
            pltpu.make_async_copy(
                ewb.at[sl, k], acc.at[dstb.at[sl, k]], ssems[sl]).wait()

    load_idx(0, 0)
    fire(0)

    def pair(p, carry):
        load_idx(2 * p - 1, 1)
        fire(1)
        drain(0)
        load_idx(2 * p, 0)
        fire(0)
        drain(1)
        return carry

    lax.fori_loop(1, NG // 2, pair, 0)
    load_idx(NG - 1, 1)
    fire(1)
    drain(0)
    drain(1)
    plsc.subcore_barrier()
    pltpu.sync_copy(acc.at[pl.ds(s * NPT, NPT)],
                    out_hbm.at[c, pl.ds(s * NPT, NPT)])


def _sc_degree(dstp, ewp):
    return pl.kernel(
        _deg_body,
        out_type=jax.ShapeDtypeStruct((NC, NPAD), jnp.float32),
        mesh=_mesh(),
        compiler_params=pltpu.CompilerParams(use_tc_tiling_on_sc=False),
        scratch_types=[
            pltpu.VMEM_SHARED((NPAD,), jnp.float32),
            pltpu.VMEM((2, G, CHUNK), jnp.int32),
            pltpu.VMEM((2, G, CHUNK), jnp.float32),
            pltpu.VMEM((NPT,), jnp.float32),
            pltpu.SemaphoreType.DMA,
            pltpu.SemaphoreType.DMA,
        ],
    )(dstp, ewp)


def _agg_body(width, gsz, src_hbm, dst_hbm, ew_hbm, g_hbm, out_hbm,
              acc, rows, srcb, dstb, ewb,
              gsem0, gsem1, ssem0, ssem1):
    G = gsz
    NG = CPT // G
    c = lax.axis_index("c")
    s = lax.axis_index("s")
    w = s * NC + c
    q = width // LANES
    gsems = (gsem0, gsem1)
    ssems = (ssem0, ssem1)


    zero = jnp.zeros((LANES,), jnp.float32)
    for i in range(CHUNK):
        for k in range(q):
            rows[0, 0, i, pl.ds(k * LANES, LANES)] = zero
    for i in range(NPT // CHUNK):
        pltpu.sync_copy(rows.at[0, 0],
                        acc.at[pl.ds(s * NPT + i * CHUNK, CHUNK)])
    plsc.subcore_barrier()

    def load_idx(grp, sl):
        row0 = w * CPT + grp * G
        pltpu.sync_copy(src_hbm.at[pl.ds(row0, G)], srcb.at[sl])
        pltpu.sync_copy(dst_hbm.at[pl.ds(row0, G)], dstb.at[sl])
        pltpu.sync_copy(ew_hbm.at[pl.ds(row0, G)], ewb.at[sl])

    def fire_gathers(sl):
        for k in range(G):
            pltpu.make_async_copy(
                g_hbm.at[srcb.at[sl, k]], rows.at[sl, k], gsems[sl]).start()

    def drain_gathers(sl):
        for k in range(G):
            pltpu.make_async_copy(
                g_hbm.at[srcb.at[sl, k]], rows.at[sl, k], gsems[sl]).wait()

    def fire_scatters(sl):
        for k in range(G):
            pltpu.make_async_copy(
                rows.at[sl, k], acc.at[dstb.at[sl, k]],
                ssems[sl]).start(add=True)

    def drain_scatters(sl):
        for k in range(G):
            pltpu.make_async_copy(
                rows.at[sl, k], acc.at[dstb.at[sl, k]], ssems[sl]).wait()

    def scale(sl):
        for k in range(G):
            def mk(_k):
                def body(e16):
                    base = e16 * LANES
                    ws = ewb[sl, _k, pl.ds(base, LANES)]
                    for i in range(LANES):
                        sc = ws[i]
                        for kk in range(q):
                            cs = pl.ds(kk * LANES, LANES)
                            rows[sl, _k, base + i, cs] = (
                                rows[sl, _k, base + i, cs] * sc)
                return body

            plsc.parallel_loop(0, CHUNK // LANES)(mk(k))

    load_idx(0, 0)
    fire_gathers(0)
    load_idx(1, 1)
    fire_gathers(1)
    drain_gathers(0)
    scale(0)
    fire_scatters(0)

    def pair(p, carry):
        drain_scatters(0)
        load_idx(2 * p, 0)
        fire_gathers(0)
        drain_gathers(1)
        scale(1)
        fire_scatters(1)
        drain_scatters(1)
        load_idx(2 * p + 1, 1)
        fire_gathers(1)
        drain_gathers(0)
        scale(0)
        fire_scatters(0)
        return carry

    lax.fori_loop(1, NG // 2, pair, 0)
    drain_gathers(1)
    scale(1)
    fire_scatters(1)
    drain_scatters(0)
    drain_scatters(1)
    plsc.subcore_barrier()
    pltpu.sync_copy(acc.at[pl.ds(s * NPT, NPT)],
                    out_hbm.at[c, pl.ds(s * NPT, NPT)])


def _sc_aggregate(srcp, dstp, ewp, g):
    width = g.shape[1]
    gsz = 5 if width > 16 else 20
    return pl.kernel(
        functools.partial(_agg_body, width, gsz),
        out_type=jax.ShapeDtypeStruct((NC, NPAD, width), jnp.float32),
        mesh=_mesh(),
        compiler_params=pltpu.CompilerParams(use_tc_tiling_on_sc=False),
        scratch_types=[
            pltpu.VMEM_SHARED((NPAD, width), jnp.float32),
            pltpu.VMEM((2, gsz, CHUNK, width), jnp.float32),
            pltpu.VMEM((2, gsz, CHUNK), jnp.int32),
            pltpu.VMEM((2, gsz, CHUNK), jnp.int32),
            pltpu.VMEM((2, gsz, CHUNK), jnp.float32),
            pltpu.SemaphoreType.DMA,
            pltpu.SemaphoreType.DMA,
            pltpu.SemaphoreType.DMA,
            pltpu.SemaphoreType.DMA,
        ],
    )(srcp, dstp, ewp, g)


def _dinv_from_parts(deg_ref):
    deg = deg_ref[0, :N] + deg_ref[1, :N] + 1.0
    return jnp.where(deg > 0.0, lax.rsqrt(deg), 0.0)


def _tc_pre_body(x_ref, w_ref, deg_ref, h_ref, g_ref):
    h = jnp.dot(x_ref[...], w_ref[...], preferred_element_type=jnp.float32)
    dinv = _dinv_from_parts(deg_ref)
    h_ref[...] = h
    g_ref[...] = h * dinv[:, None]


def _tc_layer_body(relu, deg_ref, acc_ref, h_ref, b_ref, w_ref,
                   h2_ref, g2_ref):
    dinv = _dinv_from_parts(deg_ref)
    acc = acc_ref[0, :N, :] + acc_ref[1, :N, :]
    z = acc * dinv[:, None] + h_ref[...] * (dinv * dinv)[:, None] + b_ref[...]
    if relu:
        z = jnp.maximum(z, 0.0)
    h2 = jnp.dot(z, w_ref[...], preferred_element_type=jnp.float32)
    h2_ref[...] = h2
    g2_ref[...] = h2 * dinv[:, None]


def _tc_final_body(deg_ref, acc_ref, h_ref, b_ref, out_ref):
    dinv = _dinv_from_parts(deg_ref)
    acc = acc_ref[0, :N, :] + acc_ref[1, :N, :]
    out_ref[...] = (acc * dinv[:, None]
                    + h_ref[...] * (dinv * dinv)[:, None] + b_ref[...])


def _tc_pre(x, w1, degp):
    return pl.pallas_call(
        _tc_pre_body,
        out_shape=(jax.ShapeDtypeStruct((N, HID), jnp.float32),
                   jax.ShapeDtypeStruct((N, HID), jnp.float32)),
    )(x, w1, degp)


def _tc_layer(degp, accp, h, b, w, relu=True):
    wout = w.shape[1]
    return pl.pallas_call(
        functools.partial(_tc_layer_body, relu),
        out_shape=(jax.ShapeDtypeStruct((N, wout), jnp.float32),
                   jax.ShapeDtypeStruct((N, wout), jnp.float32)),
    )(degp, accp, h, b, w)


def _tc_final(degp, accp, h, b):
    return pl.pallas_call(
        _tc_final_body,
        out_shape=jax.ShapeDtypeStruct((N, SP), jnp.float32),
    )(degp, accp, h, b)


@jax.jit
def kernel(x, edge_index, edge_weight, W1, b1, W2, b2, W3, b3):
    src = edge_index[0]
    dst = edge_index[1]

    pad = EP - E
    spread = (jnp.arange(pad, dtype=jnp.int32) * 37) % N
    srcp = jnp.concatenate([src, spread]).reshape(NROWS, CHUNK)
    dstp = jnp.concatenate([dst, spread]).reshape(NROWS, CHUNK)
    ewp = jnp.concatenate(
        [edge_weight, jnp.zeros((pad,), jnp.float32)]).reshape(NROWS, CHUNK)

    w3p = jnp.zeros((HID, SP), jnp.float32).at[:, :SOUT].set(W3)
    b3p = jnp.zeros((SP,), jnp.float32).at[:SOUT].set(b3)

    degp = _sc_degree(dstp, ewp)
    h1, g1 = _tc_pre(x, W1, degp)

    acc1 = _sc_aggregate(srcp, dstp, ewp, g1)
    h2, g2 = _tc_layer(degp, acc1, h1, b1.reshape(1, HID), W2, relu=True)

    acc2 = _sc_aggregate(srcp, dstp, ewp, g2)
    h3, g3 = _tc_layer(degp, acc2, h2, b2.reshape(1, HID), w3p, relu=True)

    acc3 = _sc_aggregate(srcp, dstp, ewp, g3)
    out = _tc_final(degp, acc3, h3, b3p.reshape(1, SP))

    return out[:, :SOUT]
